# initial kernel scaffold (unmeasured)
import jax
import jax.numpy as jnp
from jax import lax
from jax.experimental import pallas as pl
from jax.experimental.pallas import tpu as pltpu


def kernel(Q, K, V):
    b, _, h, d = Q.shape
    kv = K.shape[1]
    scale = d ** -0.5

    def body(q_ref, k_ref, v_ref, o_ref,
             o_send, o_recv, m_send, m_recv, l_send, l_recv,
             send_sems, recv_sems):
        bi = pl.program_id(0)
        my_x = lax.axis_index("x")
        my_y = lax.axis_index("y")
        my_z = lax.axis_index("z")
        peer = (my_x, 1 - my_y, my_z)

        @pl.when(bi == 0)
        def _barrier():
            bsem = pltpu.get_barrier_semaphore()
            pl.semaphore_signal(
                bsem, inc=1, device_id=peer,
                device_id_type=pl.DeviceIdType.MESH,
            )
            pl.semaphore_wait(bsem, 1)

        q = (q_ref[...].reshape(h, d) * scale).astype(jnp.bfloat16)
        row = lax.broadcasted_iota(jnp.int32, (h, h), 0)
        col = lax.broadcasted_iota(jnp.int32, (h, h), 1)
        eye = (row == col)
        qwide = (q[:, None, :] * eye[:, :, None].astype(jnp.bfloat16))
        qwide = qwide.reshape(h, h * d)

        k2 = k_ref[...].astype(jnp.bfloat16).reshape(kv, h * d)
        s = lax.dot_general(
            qwide, k2, (((1,), (1,)), ((), ())),
            preferred_element_type=jnp.float32,
        )

        m = jnp.max(s, axis=1, keepdims=True)
        p = jnp.exp(s - m)
        l = jnp.sum(p, axis=1, keepdims=True)

        v2 = v_ref[...].astype(jnp.bfloat16).reshape(kv, h * d)
        ow = lax.dot_general(
            p.astype(jnp.bfloat16), v2, (((1,), (0,)), ((), ())),
            preferred_element_type=jnp.float32,
        ).reshape(h, h, d)
        o_part = jnp.sum(ow * eye[:, :, None].astype(jnp.float32), axis=1)

        o_send[pl.ds(bi, 1)] = o_part.astype(jnp.bfloat16).reshape(1, h, d)
        m_send[pl.ds(bi, 1)] = m.reshape(1, h, 1)
        l_send[pl.ds(bi, 1)] = l.reshape(1, h, 1)

        @pl.when(bi == b - 1)
        def _exchange():
            rdmas = []
            for i, (src, dst) in enumerate(
                [(o_send, o_recv), (m_send, m_recv), (l_send, l_recv)]
            ):
                r = pltpu.make_async_remote_copy(
                    src_ref=src, dst_ref=dst,
                    send_sem=send_sems.at[i], recv_sem=recv_sems.at[i],
                    device_id=peer, device_id_type=pl.DeviceIdType.MESH,
                )
                r.start()
                rdmas.append(r)
            for r in rdmas:
                r.wait()

            oa = o_send[...].astype(jnp.float32)
            ob = o_recv[...].astype(jnp.float32)
            ma = m_send[...]
            mb = m_recv[...]
            la = l_send[...]
            lb = l_recv[...]
            mt = jnp.maximum(ma, mb)
            aa = jnp.exp(ma - mt)
            ab = jnp.exp(mb - mt)
            lt = aa * la + ab * lb
            out = (aa * oa + ab * ob) / lt
            o_ref[...] = out.reshape(b, 1, h, d)

    return pl.pallas_call(
        body,
        grid=(b,),
        out_shape=jax.ShapeDtypeStruct((b, 1, h, d), jnp.float32),
        in_specs=[
            pl.BlockSpec((1, 1, h, d), lambda i: (i, 0, 0, 0)),
            pl.BlockSpec((1, kv, h, d), lambda i: (i, 0, 0, 0)),
            pl.BlockSpec((1, kv, h, d), lambda i: (i, 0, 0, 0)),
        ],
        out_specs=pl.BlockSpec((b, 1, h, d), lambda i: (0, 0, 0, 0)),
        scratch_shapes=[
            pltpu.VMEM((b, h, d), jnp.bfloat16),
            pltpu.VMEM((b, h, d), jnp.bfloat16),
            pltpu.VMEM((b, h, 1), jnp.float32),
            pltpu.VMEM((b, h, 1), jnp.float32),
            pltpu.VMEM((b, h, 1), jnp.float32),
            pltpu.VMEM((b, h, 1), jnp.float32),
            pltpu.SemaphoreType.DMA((3,)),
            pltpu.SemaphoreType.DMA((3,)),
        ],
        compiler_params=pltpu.CompilerParams(
            collective_id=0,
            dimension_semantics=("arbitrary",),
        ),
    )(Q, K, V)


# baseline (device time: 306945 ns/iter reference)
import jax
import jax.numpy as jnp
from jax import lax
from jax.experimental import pallas as pl
from jax.experimental.pallas import tpu as pltpu


def kernel(Q, K, V):
    b, _, h, d = Q.shape
    kv = K.shape[1]
    scale = d ** -0.5

    def body(q_ref, k_ref, v_ref, o_ref,
             o_send, o_recv, m_send, m_recv, l_send, l_recv,
             send_sems, recv_sems):
        bi = pl.program_id(0)
        my_x = lax.axis_index("x")
        my_y = lax.axis_index("y")
        my_z = lax.axis_index("z")
        peer = (my_x, 1 - my_y, my_z)

        @pl.when(bi == 0)
        def _barrier():
            bsem = pltpu.get_barrier_semaphore()
            pl.semaphore_signal(
                bsem, inc=1, device_id=peer,
                device_id_type=pl.DeviceIdType.MESH,
            )
            pl.semaphore_wait(bsem, 1)

        q = (q_ref[...].reshape(h, d) * scale).astype(jnp.bfloat16)
        row = lax.broadcasted_iota(jnp.int32, (h, h, 1), 0)
        col = lax.broadcasted_iota(jnp.int32, (h, h, 1), 1)
        eye = (row == col)
        qwide = (q[:, None, :] * eye.astype(jnp.bfloat16))
        qwide = qwide.reshape(h, h * d)

        k2 = k_ref[...].astype(jnp.bfloat16).reshape(kv, h * d)
        s = lax.dot_general(
            qwide, k2, (((1,), (1,)), ((), ())),
            preferred_element_type=jnp.float32,
        )

        m = jnp.max(s, axis=1, keepdims=True)
        p = jnp.exp(s - m)
        l = jnp.sum(p, axis=1, keepdims=True)

        v2 = v_ref[...].astype(jnp.bfloat16).reshape(kv, h * d)
        ow = lax.dot_general(
            p.astype(jnp.bfloat16), v2, (((1,), (0,)), ((), ())),
            preferred_element_type=jnp.float32,
        ).reshape(h, h, d)
        o_part = jnp.sum(ow * eye.astype(jnp.float32), axis=1)

        o_send[pl.ds(bi, 1)] = o_part.astype(jnp.bfloat16).reshape(1, h, d)
        m_send[pl.ds(bi, 1)] = m.reshape(1, h, 1)
        l_send[pl.ds(bi, 1)] = l.reshape(1, h, 1)

        @pl.when(bi == b - 1)
        def _exchange():
            rdmas = []
            for i, (src, dst) in enumerate(
                [(o_send, o_recv), (m_send, m_recv), (l_send, l_recv)]
            ):
                r = pltpu.make_async_remote_copy(
                    src_ref=src, dst_ref=dst,
                    send_sem=send_sems.at[i], recv_sem=recv_sems.at[i],
                    device_id=peer, device_id_type=pl.DeviceIdType.MESH,
                )
                r.start()
                rdmas.append(r)
            for r in rdmas:
                r.wait()

            oa = o_send[...].astype(jnp.float32)
            ob = o_recv[...].astype(jnp.float32)
            ma = m_send[...]
            mb = m_recv[...]
            la = l_send[...]
            lb = l_recv[...]
            mt = jnp.maximum(ma, mb)
            aa = jnp.exp(ma - mt)
            ab = jnp.exp(mb - mt)
            lt = aa * la + ab * lb
            out = (aa * oa + ab * ob) / lt
            o_ref[...] = out.reshape(b, 1, h, d)

    return pl.pallas_call(
        body,
        grid=(b,),
        out_shape=jax.ShapeDtypeStruct((b, 1, h, d), jnp.float32),
        in_specs=[
            pl.BlockSpec((1, 1, h, d), lambda i: (i, 0, 0, 0)),
            pl.BlockSpec((1, kv, h, d), lambda i: (i, 0, 0, 0)),
            pl.BlockSpec((1, kv, h, d), lambda i: (i, 0, 0, 0)),
        ],
        out_specs=pl.BlockSpec((b, 1, h, d), lambda i: (0, 0, 0, 0)),
        scratch_shapes=[
            pltpu.VMEM((b, h, d), jnp.bfloat16),
            pltpu.VMEM((b, h, d), jnp.bfloat16),
            pltpu.VMEM((b, h, 1), jnp.float32),
            pltpu.VMEM((b, h, 1), jnp.float32),
            pltpu.VMEM((b, h, 1), jnp.float32),
            pltpu.VMEM((b, h, 1), jnp.float32),
            pltpu.SemaphoreType.DMA((3,)),
            pltpu.SemaphoreType.DMA((3,)),
        ],
        compiler_params=pltpu.CompilerParams(
            collective_id=0,
            dimension_semantics=("arbitrary",),
            vmem_limit_bytes=100 * 1024 * 1024,
        ),
    )(Q, K, V)


# device time: 304338 ns/iter; 1.0086x vs baseline; 1.0086x over previous
import jax
import jax.numpy as jnp
from jax import lax
from jax.experimental import pallas as pl
from jax.experimental.pallas import tpu as pltpu


def kernel(Q, K, V):
    b, _, h, d = Q.shape
    kv = K.shape[1]
    scale = d ** -0.5

    def body(q_ref, k_ref, v_ref, o_ref,
             o_send, o_recv, m_send, m_recv, l_send, l_recv,
             send_sems, recv_sems):
        bi = pl.program_id(0)
        my_x = lax.axis_index("x")
        my_y = lax.axis_index("y")
        my_z = lax.axis_index("z")
        peer = (my_x, 1 - my_y, my_z)

        @pl.when(bi == 0)
        def _barrier():
            bsem = pltpu.get_barrier_semaphore()
            pl.semaphore_signal(
                bsem, inc=1, device_id=peer,
                device_id_type=pl.DeviceIdType.MESH,
            )
            pl.semaphore_wait(bsem, 1)

        qs = q_ref[...].reshape(h, d) * scale
        qsT = qs.T
        mask3 = (
            lax.broadcasted_iota(jnp.int32, (h, d, h), 0)
            == lax.broadcasted_iota(jnp.int32, (h, d, h), 2)
        )
        qwT = (qsT[None, :, :] * mask3.astype(jnp.float32))
        qwT = qwT.reshape(h * d, h).astype(jnp.bfloat16)

        k2 = k_ref[...].astype(jnp.bfloat16).reshape(kv, h * d)
        st = lax.dot_general(
            k2, qwT, (((1,), (0,)), ((), ())),
            preferred_element_type=jnp.float32,
        )

        m_row = jnp.max(st, axis=0, keepdims=True)
        p = jnp.exp(st - m_row)
        l_row = jnp.sum(p, axis=0, keepdims=True)
        pT = p.astype(jnp.bfloat16).T

        v2 = v_ref[...].astype(jnp.bfloat16).reshape(kv, h * d)
        ow = lax.dot_general(
            pT, v2, (((1,), (0,)), ((), ())),
            preferred_element_type=jnp.float32,
        ).reshape(h, h, d)
        eye = (
            lax.broadcasted_iota(jnp.int32, (h, h, 1), 0)
            == lax.broadcasted_iota(jnp.int32, (h, h, 1), 1)
        )
        o_part = jnp.sum(ow * eye.astype(jnp.float32), axis=1)

        o_send[pl.ds(bi, 1)] = o_part.astype(jnp.bfloat16).reshape(1, h, d)
        m_send[pl.ds(bi, 1)] = m_row.T.reshape(1, h, 1)
        l_send[pl.ds(bi, 1)] = l_row.T.reshape(1, h, 1)

        @pl.when(bi == b - 1)
        def _exchange():
            rdmas = []
            for i, (src, dst) in enumerate(
                [(o_send, o_recv), (m_send, m_recv), (l_send, l_recv)]
            ):
                r = pltpu.make_async_remote_copy(
                    src_ref=src, dst_ref=dst,
                    send_sem=send_sems.at[i], recv_sem=recv_sems.at[i],
                    device_id=peer, device_id_type=pl.DeviceIdType.MESH,
                )
                r.start()
                rdmas.append(r)
            for r in rdmas:
                r.wait()

            oa = o_send[...].astype(jnp.float32)
            ob = o_recv[...].astype(jnp.float32)
            ma = m_send[...]
            mb = m_recv[...]
            la = l_send[...]
            lb = l_recv[...]
            mt = jnp.maximum(ma, mb)
            aa = jnp.exp(ma - mt)
            ab = jnp.exp(mb - mt)
            lt = aa * la + ab * lb
            out = (aa * oa + ab * ob) / lt
            o_ref[...] = out.reshape(b, 1, h, d)

    return pl.pallas_call(
        body,
        grid=(b,),
        out_shape=jax.ShapeDtypeStruct((b, 1, h, d), jnp.float32),
        in_specs=[
            pl.BlockSpec((1, 1, h, d), lambda i: (i, 0, 0, 0)),
            pl.BlockSpec((1, kv, h, d), lambda i: (i, 0, 0, 0)),
            pl.BlockSpec((1, kv, h, d), lambda i: (i, 0, 0, 0)),
        ],
        out_specs=pl.BlockSpec((b, 1, h, d), lambda i: (0, 0, 0, 0)),
        scratch_shapes=[
            pltpu.VMEM((b, h, d), jnp.bfloat16),
            pltpu.VMEM((b, h, d), jnp.bfloat16),
            pltpu.VMEM((b, h, 1), jnp.float32),
            pltpu.VMEM((b, h, 1), jnp.float32),
            pltpu.VMEM((b, h, 1), jnp.float32),
            pltpu.VMEM((b, h, 1), jnp.float32),
            pltpu.SemaphoreType.DMA((3,)),
            pltpu.SemaphoreType.DMA((3,)),
        ],
        compiler_params=pltpu.CompilerParams(
            collective_id=0,
            dimension_semantics=("arbitrary",),
            vmem_limit_bytes=100 * 1024 * 1024,
        ),
    )(Q, K, V)


# device time: 74762 ns/iter; 4.1056x vs baseline; 4.0708x over previous
import jax
import jax.numpy as jnp
from jax import lax
from jax.experimental import pallas as pl
from jax.experimental.pallas import tpu as pltpu

N_STAGE = 3


def kernel(Q, K, V):
    b, _, h, d = Q.shape
    kv = K.shape[1]
    n_rep = 8
    kvc = kv // n_rep
    scale = d ** -0.5

    start = (lax.axis_index("x") * 4 + lax.axis_index("z")) * kvc
    K = lax.dynamic_slice(K, (0, start, 0, 0), (b, kvc, h, d))
    V = lax.dynamic_slice(V, (0, start, 0, 0), (b, kvc, h, d))
    K = K.reshape(b, kvc, h * d)
    V = V.reshape(b, kvc, h * d)

    def body(q_ref, k_ref, v_ref, o_ref,
             o_send, m_send, l_send, o_recv, m_recv, l_recv,
             send_sems, recv_sems):
        bi = pl.program_id(0)
        my_x = lax.axis_index("x")
        my_y = lax.axis_index("y")
        my_z = lax.axis_index("z")
        partners = [
            (1 - my_x, my_y, my_z),
            (my_x, 1 - my_y, my_z),
            (my_x, my_y, my_z ^ 1),
            (my_x, my_y, my_z ^ 2),
        ]

        @pl.when(bi == 0)
        def _barrier():
            bsem = pltpu.get_barrier_semaphore()
            for p in partners:
                pl.semaphore_signal(
                    bsem, inc=1, device_id=p,
                    device_id_type=pl.DeviceIdType.MESH,
                )
            pl.semaphore_wait(bsem, len(partners))

        qs = q_ref[...].reshape(h, d) * scale
        qsT = qs.T
        mask3 = (
            lax.broadcasted_iota(jnp.int32, (h, d, h), 0)
            == lax.broadcasted_iota(jnp.int32, (h, d, h), 2)
        )
        qwT = (qsT[None, :, :] * mask3.astype(jnp.float32))
        qwT = qwT.reshape(h * d, h).astype(jnp.bfloat16)

        k2 = k_ref[...].astype(jnp.bfloat16).reshape(kvc, h * d)
        st = lax.dot_general(
            k2, qwT, (((1,), (0,)), ((), ())),
            preferred_element_type=jnp.float32,
        )

        m_row = jnp.max(st, axis=0, keepdims=True)
        p = jnp.exp(st - m_row)
        l_row = jnp.sum(p, axis=0, keepdims=True)
        pT = p.astype(jnp.bfloat16).T

        v2 = v_ref[...].astype(jnp.bfloat16).reshape(kvc, h * d)
        ow = lax.dot_general(
            pT, v2, (((1,), (0,)), ((), ())),
            preferred_element_type=jnp.float32,
        ).reshape(h, h, d)
        eye = (
            lax.broadcasted_iota(jnp.int32, (h, h, 1), 0)
            == lax.broadcasted_iota(jnp.int32, (h, h, 1), 1)
        )
        o_part = jnp.sum(ow * eye.astype(jnp.float32), axis=1)

        o_send[pl.ds(bi, 1)] = o_part.reshape(1, h, d)
        m_send[pl.ds(bi, 1)] = m_row.T.reshape(1, h, 1)
        l_send[pl.ds(bi, 1)] = l_row.T.reshape(1, h, 1)

        @pl.when(bi == b - 1)
        def _allreduce():
            for s, partner in enumerate(partners):
                rdmas = []
                for i, (src, dst) in enumerate(
                    [(o_send, o_recv), (m_send, m_recv), (l_send, l_recv)]
                ):
                    r = pltpu.make_async_remote_copy(
                        src_ref=src, dst_ref=dst.at[s],
                        send_sem=send_sems.at[i],
                        recv_sem=recv_sems.at[s * N_STAGE + i],
                        device_id=partner,
                        device_id_type=pl.DeviceIdType.MESH,
                    )
                    r.start()
                    rdmas.append(r)
                for r in rdmas:
                    r.wait()

                ma = m_send[...]
                mb = m_recv[s]
                mt = jnp.maximum(ma, mb)
                aa = jnp.exp(ma - mt)
                ab = jnp.exp(mb - mt)
                o_send[...] = aa * o_send[...] + ab * o_recv[s]
                l_send[...] = aa * l_send[...] + ab * l_recv[s]
                m_send[...] = mt

            out = o_send[...] / l_send[...]
            o_ref[...] = out.reshape(b, 1, h, d)

    return pl.pallas_call(
        body,
        grid=(b,),
        out_shape=jax.ShapeDtypeStruct((b, 1, h, d), jnp.float32),
        in_specs=[
            pl.BlockSpec((1, 1, h, d), lambda i: (i, 0, 0, 0)),
            pl.BlockSpec((1, kvc, h * d), lambda i: (i, 0, 0)),
            pl.BlockSpec((1, kvc, h * d), lambda i: (i, 0, 0)),
        ],
        out_specs=pl.BlockSpec((b, 1, h, d), lambda i: (0, 0, 0, 0)),
        scratch_shapes=[
            pltpu.VMEM((b, h, d), jnp.float32),
            pltpu.VMEM((b, h, 1), jnp.float32),
            pltpu.VMEM((b, h, 1), jnp.float32),
            pltpu.VMEM((4, b, h, d), jnp.float32),
            pltpu.VMEM((4, b, h, 1), jnp.float32),
            pltpu.VMEM((4, b, h, 1), jnp.float32),
            pltpu.SemaphoreType.DMA((N_STAGE,)),
            pltpu.SemaphoreType.DMA((4 * N_STAGE,)),
        ],
        compiler_params=pltpu.CompilerParams(
            collective_id=0,
            dimension_semantics=("arbitrary",),
            vmem_limit_bytes=60 * 1024 * 1024,
        ),
    )(Q, K, V)
